# baseline (speedup 1.0000x reference)
import functools

import jax
import jax.numpy as jnp
from jax import lax
from jax.experimental import pallas as pl
from jax.experimental.pallas import tpu as pltpu
from jax.experimental.pallas import tpu_sc as plsc

N = 10000
E = 320000
D = 128
ED = 16

NC = 2
NS = 16
NW = NC * NS
CH = 128
NCHUNK = 79
EPW = CH * NCHUNK
EPAD = EPW * NW
NPAD = 10240
RPT = NPAD // NS


def _edge_mlp_body(ea_ref, we_ref, be_ref, o_ref):
    o_ref[...] = (
        jnp.dot(ea_ref[...], we_ref[...], preferred_element_type=jnp.float32)
        + be_ref[...]
    )


def _edge_mlp(ea, we, be):
    m = 2048
    grid = EPAD // m
    return pl.pallas_call(
        _edge_mlp_body,
        grid=(grid,),
        in_specs=[
            pl.BlockSpec((m, ED), lambda i: (i, 0)),
            pl.BlockSpec((ED, D), lambda i: (0, 0)),
            pl.BlockSpec((1, D), lambda i: (0, 0)),
        ],
        out_specs=pl.BlockSpec((m, D), lambda i: (i, 0)),
        out_shape=jax.ShapeDtypeStruct((EPAD, D), jnp.float32),
    )(ea, we, be)


def _sc_body(h_hbm, e_hbm, sd_hbm, out_hbm,
             isd, ebuf, hbuf, acc, sem_g, sem_e):
    c = lax.axis_index("c")
    s = lax.axis_index("s")
    wid = c * NS + s
    base = wid * EPW

    zero = jnp.zeros((16,), jnp.float32)

    def _zrow(r, carry):
        for j in range(D // 16):
            hbuf[r, pl.ds(j * 16, 16)] = zero
        return carry

    lax.fori_loop(0, CH, _zrow, 0)
    for t in range(RPT // CH):
        pltpu.sync_copy(hbuf, acc.at[pl.ds(s * RPT + t * CH, CH)])
    plsc.subcore_barrier()

    def _chunk(g, carry):
        off = base + g * CH
        pltpu.sync_copy(sd_hbm.at[wid, g], isd)
        cp_g = pltpu.async_copy(h_hbm.at[isd.at[0]], hbuf, sem_g)
        cp_e = pltpu.async_copy(e_hbm.at[pl.ds(off, CH)], ebuf, sem_e)
        cp_g.wait()
        cp_e.wait()

        @plsc.parallel_loop(0, CH, unroll=2)
        def _row(r):
            for j in range(D // 16):
                sl = pl.ds(j * 16, 16)
                ebuf[r, sl] = jnp.maximum(hbuf[r, sl] + ebuf[r, sl], 0.0)
        pltpu.sync_copy(ebuf, acc.at[isd.at[1]], add=True)
        return carry

    lax.fori_loop(0, NCHUNK, _chunk, 0)
    plsc.subcore_barrier()
    for t in range(RPT // CH):
        r0 = s * RPT + t * CH
        pltpu.sync_copy(acc.at[pl.ds(r0, CH)], out_hbm.at[c, pl.ds(r0, CH)])


_sc_message = functools.partial(
    pl.kernel,
    out_type=jax.ShapeDtypeStruct((NC, NPAD, D), jnp.float32),
    mesh=plsc.VectorSubcoreMesh(core_axis_name="c", subcore_axis_name="s"),
    scratch_types=[
        pltpu.VMEM((2, CH), jnp.int32),
        pltpu.VMEM((CH, D), jnp.float32),
        pltpu.VMEM((CH, D), jnp.float32),
        pltpu.VMEM_SHARED((NPAD, D), jnp.float32),
        pltpu.SemaphoreType.DMA,
        pltpu.SemaphoreType.DMA,
    ],
)(_sc_body)


def _node_mlp_body(h_ref, p_ref, w1_ref, b1_ref, w2_ref, b2_ref, o_ref):
    z = h_ref[...] + p_ref[0] + p_ref[1]
    t = jnp.maximum(
        jnp.dot(z, w1_ref[...], preferred_element_type=jnp.float32) + b1_ref[...],
        0.0,
    )
    o = jnp.dot(t, w2_ref[...], preferred_element_type=jnp.float32) + b2_ref[...]
    o_ref[...] = jnp.maximum(o, 0.0)


def _node_mlp(h, parts, w1, b1, w2, b2):
    m = 1024
    grid = (N + m - 1) // m
    return pl.pallas_call(
        _node_mlp_body,
        grid=(grid,),
        in_specs=[
            pl.BlockSpec((m, D), lambda i: (i, 0)),
            pl.BlockSpec((NC, m, D), lambda i: (0, i, 0)),
            pl.BlockSpec((D, D), lambda i: (0, 0)),
            pl.BlockSpec((1, D), lambda i: (0, 0)),
            pl.BlockSpec((D, D), lambda i: (0, 0)),
            pl.BlockSpec((1, D), lambda i: (0, 0)),
        ],
        out_specs=pl.BlockSpec((m, D), lambda i: (i, 0)),
        out_shape=jax.ShapeDtypeStruct((N, D), jnp.float32),
    )(h, parts, w1, b1, w2, b2)


def _final_body(h_ref, wt_ref, bt_ref, o_ref):
    t = jnp.sum(h_ref[...] * wt_ref[...], axis=1, keepdims=True) + bt_ref[...]
    o_ref[...] = 1.0 / (1.0 + jnp.exp(-t))


def _final(h, wt, bt):
    m = 1024
    grid = (N + m - 1) // m
    return pl.pallas_call(
        _final_body,
        grid=(grid,),
        in_specs=[
            pl.BlockSpec((m, D), lambda i: (i, 0)),
            pl.BlockSpec((1, D), lambda i: (0, 0)),
            pl.BlockSpec((1, 1), lambda i: (0, 0)),
        ],
        out_specs=pl.BlockSpec((m, 1), lambda i: (i, 0)),
        out_shape=jax.ShapeDtypeStruct((N, 1), jnp.float32),
    )(h, wt, bt)


def kernel(x, edge_index, edge_attr,
           We0, be0, W10, b10, W20, b20,
           We1, be1, W11, b11, W21, b21,
           We2, be2, W12, b12, W22, b22,
           Wout, bout):
    pad = EPAD - E
    src = jnp.concatenate([edge_index[0], jnp.zeros((pad,), jnp.int32)])
    dst = jnp.concatenate([edge_index[1], jnp.full((pad,), N, jnp.int32)])
    sd = jnp.stack([src.reshape(NW, NCHUNK, CH), dst.reshape(NW, NCHUNK, CH)],
                   axis=2)
    ea = jnp.concatenate([edge_attr, jnp.zeros((pad, ED), jnp.float32)])

    layers = [(We0, be0, W10, b10, W20, b20),
              (We1, be1, W11, b11, W21, b21),
              (We2, be2, W12, b12, W22, b22)]
    h = x
    for (we, be, w1, b1, w2, b2) in layers:
        e = _edge_mlp(ea, we, be.reshape(1, D))
        parts = _sc_message(h, e, sd)
        h = _node_mlp(h, parts, w1, b1.reshape(1, D), w2, b2.reshape(1, D))
    out = _final(h, Wout.reshape(1, D), bout.reshape(1, 1))
    return jnp.squeeze(out, axis=-1)
